# baseline (speedup 1.0000x reference)
import functools

import jax
import jax.numpy as jnp
import numpy as np
from jax.experimental import pallas as pl
from jax.experimental.pallas import tpu as pltpu

B = 4096
N_OBJ = 10
N_VAL = 50
N_PER = 61
NP = 64
E = 128
H = 128
HEADS = 4
DH = H // HEADS
HS = HEADS * NP
AW = 2 * HEADS
G = 128


def _leaky(x):
    return jnp.maximum(x, 0.2 * x)


def _gat_block(xpav, A4_list, sel_den, rexp, gh1, colmask, bias):
    xp2 = xpav[:, :H]
    av2 = xpav[:, H:H + AW]
    asrc3 = av2[:, :HEADS].reshape(G, NP, HEADS)
    adst2 = av2[:, HEADS:]
    u = jnp.transpose(asrc3, (0, 2, 1)).reshape(G, HS)
    c = _leaky(2.0 * jnp.max(av2))
    rp = jnp.concatenate([rexp, u], axis=0)
    lhs = jnp.concatenate([adst2, gh1], axis=1)
    rd = jax.lax.dot(lhs, rp)
    ex = jnp.exp(_leaky(rd) - c)

    bf = jnp.bfloat16
    xpb = xp2.astype(bf)
    xstk_all = (xpb.reshape(G, 1, NP, H) * colmask[None].astype(bf)
                ).reshape(G, HS, H)
    exb = ex.astype(bf)
    ex3 = exb.reshape(G, NP, HS)
    outs = []
    for g in range(G):
        Eg = ex3[g] * A4_list[g].astype(bf)
        res = jax.lax.dot(
            Eg, jnp.concatenate([xstk_all[g], sel_den.astype(bf)], axis=1),
            preferred_element_type=jnp.float32)
        outs.append(res[:, :H] / res[:, H:] + bias)
    return jnp.concatenate(outs, axis=0)


def _fused_kernel(feat_ref, ei_ref, wcat_ref, brow_ref,
                  wa0_ref, b0_ref, wa1_ref, b1_ref,
                  sel_den_ref, rexp_ref, gh1_ref, colmask_ref, eye4_ref,
                  outh_ref, outv_ref):
    x2 = jax.lax.dot_general(
        feat_ref[...], wcat_ref[...], (((0,), (0,)), ((), ())))
    x2 = (x2.reshape(G, NP, H) + brow_ref[...][None]).reshape(G * NP, H)
    x2 = jnp.maximum(x2, 0.0)

    col_np = jax.lax.broadcasted_iota(jnp.int32, (NP, 1), 0)
    col_hs = jax.lax.broadcasted_iota(jnp.int32, (HS, 1), 0) % NP

    A4_list = []
    for g in range(G):
        src = ei_ref[g][0:1, :]
        dst = ei_ref[g][1:2, :]
        oh_src4T = jnp.float32(src == col_hs)
        oh_dstT = jnp.float32(dst == col_np)
        A4 = jax.lax.dot_general(
            oh_dstT, oh_src4T, (((1,), (1,)), ((), ())))
        A4_list.append(A4 + eye4_ref[...])

    sel_den = sel_den_ref[...]
    rexp = rexp_ref[...]
    gh1 = gh1_ref[...]
    colmask = colmask_ref[...][:, None, :]

    def layer(x2, wa_ref, b_ref):
        xpav = jax.lax.dot(x2, wa_ref[...])
        return _gat_block(xpav, A4_list, sel_den, rexp, gh1, colmask,
                          b_ref[...])

    h1 = jnp.maximum(layer(x2, wa0_ref, b0_ref), 0.0)
    h2 = jnp.maximum(layer(h1, wa1_ref, b1_ref), 0.0)
    h3 = h2.reshape(G, NP, H)
    outh_ref[...] = h3[:, 0, :]
    outv_ref[...] = h3[:, N_PER - N_VAL:N_PER, :]


@jax.jit
def _run(feat, ei, wcat, brow, wa0, b0, wa1, b1,
         sel_den, rexp, gh1, colmask, eye4):
    grid = (B // G,)
    full = lambda *s: pl.BlockSpec(s, lambda i: tuple(0 for _ in s))
    return pl.pallas_call(
        _fused_kernel,
        grid=grid,
        in_specs=[
            pl.BlockSpec((9, G * NP), lambda i: (0, i)),
            pl.BlockSpec((G, 2, E), lambda i: (i, 0, 0)),
            full(9, H),
            full(NP, H),
            full(H, H + AW),
            full(1, H),
            full(H, H + AW),
            full(1, H),
            full(HS, H),
            full(HEADS, HS),
            full(G * NP, G),
            full(HEADS, H),
            full(NP, HS),
        ],
        out_specs=[
            pl.BlockSpec((G, H), lambda i: (i, 0)),
            pl.BlockSpec((G, N_VAL, H), lambda i: (i, 0, 0)),
        ],
        out_shape=[
            jax.ShapeDtypeStruct((B, H), jnp.float32),
            jax.ShapeDtypeStruct((B, N_VAL, H), jnp.float32),
        ],
    )(feat, ei, wcat, brow, wa0, b0, wa1, b1,
      sel_den, rexp, gh1, colmask, eye4)


def kernel(head_node, objective_nodes, value_nodes, edge_indices,
           W_head, b_head, W_obj, b_obj, W_val, b_val,
           W0, att_src0, att_dst0, bias0,
           W1, att_src1, att_dst1, bias1):
    f32 = jnp.float32
    feat3 = jnp.zeros((9, B, NP), f32)
    feat3 = feat3.at[0:2, :, 0].set(jnp.transpose(head_node))
    feat3 = feat3.at[2:4, :, 1:1 + N_OBJ].set(
        jnp.transpose(objective_nodes, (2, 0, 1)))
    feat3 = feat3.at[4:9, :, 1 + N_OBJ:N_PER].set(
        jnp.transpose(value_nodes, (2, 0, 1)))
    feat = feat3.reshape(9, B * NP)
    wcat = jnp.concatenate([W_head, W_obj, W_val], axis=0)
    brow = jnp.concatenate([
        b_head[None, :],
        jnp.tile(b_obj[None, :], (N_OBJ, 1)),
        jnp.tile(b_val[None, :], (N_VAL, 1)),
        jnp.zeros((NP - N_PER, H), f32),
    ], axis=0)
    att0 = jnp.zeros((H, AW), f32)
    att1 = jnp.zeros((H, AW), f32)
    for h in range(HEADS):
        att0 = att0.at[h * DH:(h + 1) * DH, h].set(att_src0[h])
        att0 = att0.at[h * DH:(h + 1) * DH, HEADS + h].set(att_dst0[h])
        att1 = att1.at[h * DH:(h + 1) * DH, h].set(att_src1[h])
        att1 = att1.at[h * DH:(h + 1) * DH, HEADS + h].set(att_dst1[h])
    wa0 = jnp.concatenate([W0, W0 @ att0], axis=1)
    wa1 = jnp.concatenate([W1, W1 @ att1], axis=1)
    ii = np.arange(HS)
    sel_den = jnp.asarray((ii[:, None] // NP) == (np.arange(H)[None] // DH),
                          f32)
    rexp = jnp.asarray(np.arange(HEADS)[:, None] == (ii[None] // NP), f32)
    gh1 = jnp.asarray((np.arange(G * NP)[:, None] // NP)
                      == np.arange(G)[None], f32)
    colmask = jnp.asarray(np.arange(HEADS)[:, None]
                          == (np.arange(H)[None] // DH), f32)
    eye4 = jnp.asarray(np.arange(NP)[:, None] == (ii[None] % NP), f32)
    outh, outv = _run(feat, edge_indices, wcat, brow,
                      wa0, bias0[None, :], wa1, bias1[None, :],
                      sel_den, rexp, gh1, colmask, eye4)
    return (outh, outv)


# speedup vs baseline: 1.0446x; 1.0446x over previous
import functools

import jax
import jax.numpy as jnp
import numpy as np
from jax.experimental import pallas as pl
from jax.experimental.pallas import tpu as pltpu

B = 4096
N_OBJ = 10
N_VAL = 50
N_PER = 61
NP = 64
E = 128
H = 128
HEADS = 4
DH = H // HEADS
HS = HEADS * NP
AW = 2 * HEADS
G = 128


def _leaky(x):
    return jnp.maximum(x, 0.2 * x)


def _gat_block(xpav, A4_list, sel_den, rexp, gh1, colmask, bias):
    xp2 = xpav[:, :H]
    av2 = xpav[:, H:H + AW]
    asrc3 = av2[:, :HEADS].reshape(G, NP, HEADS)
    adst2 = av2[:, HEADS:]
    u = jnp.transpose(asrc3, (0, 2, 1)).reshape(G, HS)
    c = _leaky(2.0 * jnp.max(av2))
    rp = jnp.concatenate([rexp, u], axis=0)
    lhs = jnp.concatenate([adst2, gh1], axis=1)
    rd = jax.lax.dot(lhs, rp)
    ex = jnp.exp(_leaky(rd) - c)

    xstk_all = (xp2.reshape(G, 1, NP, H) * colmask[None]).reshape(G, HS, H)
    ex3 = ex.reshape(G, NP, HS)
    outs = []
    for g in range(G):
        Eg = ex3[g] * A4_list[g]
        res = jax.lax.dot(
            Eg, jnp.concatenate([xstk_all[g], sel_den], axis=1))
        outs.append(res[:, :H] / res[:, H:] + bias)
    return jnp.concatenate(outs, axis=0)


def _fused_kernel(feat_ref, ei_ref, wcat_ref, brow_ref,
                  wa0_ref, b0_ref, wa1_ref, b1_ref,
                  sel_den_ref, rexp_ref, gh1_ref, colmask_ref, eye4_ref,
                  outh_ref, outv_ref):
    x2 = jax.lax.dot_general(
        feat_ref[...], wcat_ref[...], (((0,), (0,)), ((), ())))
    x2 = (x2.reshape(G, NP, H) + brow_ref[...][None]).reshape(G * NP, H)
    x2 = jnp.maximum(x2, 0.0)

    col_np = jax.lax.broadcasted_iota(jnp.int32, (NP, 1), 0)
    col_hs = jax.lax.broadcasted_iota(jnp.int32, (HS, 1), 0) % NP

    A4_list = []
    for g in range(G):
        src = ei_ref[g][0:1, :]
        dst = ei_ref[g][1:2, :]
        oh_src4T = jnp.float32(src == col_hs)
        oh_dstT = jnp.float32(dst == col_np)
        A4 = jax.lax.dot_general(
            oh_dstT, oh_src4T, (((1,), (1,)), ((), ())))
        A4_list.append(A4 + eye4_ref[...])

    sel_den = sel_den_ref[...]
    rexp = rexp_ref[...]
    gh1 = gh1_ref[...]
    colmask = colmask_ref[...][:, None, :]

    def layer(x2, wa_ref, b_ref):
        xpav = jax.lax.dot(x2, wa_ref[...])
        return _gat_block(xpav, A4_list, sel_den, rexp, gh1, colmask,
                          b_ref[...])

    h1 = jnp.maximum(layer(x2, wa0_ref, b0_ref), 0.0)
    h2 = jnp.maximum(layer(h1, wa1_ref, b1_ref), 0.0)
    h3 = h2.reshape(G, NP, H)
    outh_ref[...] = h3[:, 0, :]
    outv_ref[...] = h3[:, N_PER - N_VAL:N_PER, :]


@jax.jit
def _run(feat, ei, wcat, brow, wa0, b0, wa1, b1,
         sel_den, rexp, gh1, colmask, eye4):
    grid = (B // G,)
    full = lambda *s: pl.BlockSpec(s, lambda i: tuple(0 for _ in s))
    return pl.pallas_call(
        _fused_kernel,
        grid=grid,
        in_specs=[
            pl.BlockSpec((9, G * NP), lambda i: (0, i)),
            pl.BlockSpec((G, 2, E), lambda i: (i, 0, 0)),
            full(9, H),
            full(NP, H),
            full(H, H + AW),
            full(1, H),
            full(H, H + AW),
            full(1, H),
            full(HS, H),
            full(HEADS, HS),
            full(G * NP, G),
            full(HEADS, H),
            full(NP, HS),
        ],
        out_specs=[
            pl.BlockSpec((G, H), lambda i: (i, 0)),
            pl.BlockSpec((G, N_VAL, H), lambda i: (i, 0, 0)),
        ],
        out_shape=[
            jax.ShapeDtypeStruct((B, H), jnp.float32),
            jax.ShapeDtypeStruct((B, N_VAL, H), jnp.float32),
        ],
    )(feat, ei, wcat, brow, wa0, b0, wa1, b1,
      sel_den, rexp, gh1, colmask, eye4)


def kernel(head_node, objective_nodes, value_nodes, edge_indices,
           W_head, b_head, W_obj, b_obj, W_val, b_val,
           W0, att_src0, att_dst0, bias0,
           W1, att_src1, att_dst1, bias1):
    f32 = jnp.float32
    feat3 = jnp.zeros((9, B, NP), f32)
    feat3 = feat3.at[0:2, :, 0].set(jnp.transpose(head_node))
    feat3 = feat3.at[2:4, :, 1:1 + N_OBJ].set(
        jnp.transpose(objective_nodes, (2, 0, 1)))
    feat3 = feat3.at[4:9, :, 1 + N_OBJ:N_PER].set(
        jnp.transpose(value_nodes, (2, 0, 1)))
    feat = feat3.reshape(9, B * NP)
    wcat = jnp.concatenate([W_head, W_obj, W_val], axis=0)
    brow = jnp.concatenate([
        b_head[None, :],
        jnp.tile(b_obj[None, :], (N_OBJ, 1)),
        jnp.tile(b_val[None, :], (N_VAL, 1)),
        jnp.zeros((NP - N_PER, H), f32),
    ], axis=0)
    att0 = jnp.zeros((H, AW), f32)
    att1 = jnp.zeros((H, AW), f32)
    for h in range(HEADS):
        att0 = att0.at[h * DH:(h + 1) * DH, h].set(att_src0[h])
        att0 = att0.at[h * DH:(h + 1) * DH, HEADS + h].set(att_dst0[h])
        att1 = att1.at[h * DH:(h + 1) * DH, h].set(att_src1[h])
        att1 = att1.at[h * DH:(h + 1) * DH, HEADS + h].set(att_dst1[h])
    wa0 = jnp.concatenate([W0, W0 @ att0], axis=1)
    wa1 = jnp.concatenate([W1, W1 @ att1], axis=1)
    ii = np.arange(HS)
    sel_den = jnp.asarray((ii[:, None] // NP) == (np.arange(H)[None] // DH),
                          f32)
    rexp = jnp.asarray(np.arange(HEADS)[:, None] == (ii[None] // NP), f32)
    gh1 = jnp.asarray((np.arange(G * NP)[:, None] // NP)
                      == np.arange(G)[None], f32)
    colmask = jnp.asarray(np.arange(HEADS)[:, None]
                          == (np.arange(H)[None] // DH), f32)
    eye4 = jnp.asarray(np.arange(NP)[:, None] == (ii[None] % NP), f32)
    outh, outv = _run(feat, edge_indices, wcat, brow,
                      wa0, bias0[None, :], wa1, bias1[None, :],
                      sel_den, rexp, gh1, colmask, eye4)
    return (outh, outv)
